# baseline (speedup 1.0000x reference)
import functools

import jax
import jax.numpy as jnp
from jax import lax
from jax.experimental import pallas as pl
from jax.experimental.pallas import tpu as pltpu
from jax.experimental.pallas import tpu_sc as plsc

DIM = 64
GROUP = 128
NUM_WORKERS = 32


def _gather_body(table_hbm, idx_hbm, out_hbm, idx_v, rows_v, sem):
  ngroups = idx_hbm.shape[1]
  wid = lax.axis_index("s") * 2 + lax.axis_index("c")
  pltpu.sync_copy(idx_hbm.at[wid], idx_v)
  base = wid * (ngroups * GROUP)

  def body(g, carry):
    pltpu.async_copy(table_hbm.at[idx_v.at[g]], rows_v, sem).wait()
    pltpu.sync_copy(rows_v, out_hbm.at[pl.ds(base + g * GROUP, GROUP)])
    return carry

  lax.fori_loop(0, ngroups, body, 0)


def _sc_gather(table, idx3):
  nw, ngroups, group = idx3.shape
  n = nw * ngroups * group
  mesh = plsc.VectorSubcoreMesh(core_axis_name="c", subcore_axis_name="s")
  return pl.kernel(
      _gather_body,
      out_type=jax.ShapeDtypeStruct((n, DIM), jnp.float32),
      mesh=mesh,
      scratch_types=[
          pltpu.VMEM((ngroups, group), jnp.int32),
          pltpu.VMEM((group, DIM), jnp.float32),
          pltpu.SemaphoreType.DMA,
      ],
      compiler_params=pltpu.CompilerParams(use_tc_tiling_on_sc=False),
  )(table, idx3)


def _mlp_body(emb_ref, we_ref, be_ref, wd_ref, bd_ref, out_ref):
  h = jnp.dot(emb_ref[...], we_ref[...], preferred_element_type=jnp.float32)
  h = jnp.maximum(h + be_ref[...], 0.0)
  out = jnp.dot(h, wd_ref[...], preferred_element_type=jnp.float32)
  out_ref[...] = out + bd_ref[...]


def _tc_mlp(emb, w_enc, b_enc, w_dec, b_dec, blk):
  n = emb.shape[0]
  return pl.pallas_call(
      _mlp_body,
      grid=(n // blk,),
      in_specs=[
          pl.BlockSpec((blk, DIM), lambda i: (i, 0)),
          pl.BlockSpec((DIM, DIM), lambda i: (0, 0)),
          pl.BlockSpec((1, DIM), lambda i: (0, 0)),
          pl.BlockSpec((DIM, DIM), lambda i: (0, 0)),
          pl.BlockSpec((1, DIM), lambda i: (0, 0)),
      ],
      out_specs=pl.BlockSpec((blk, DIM), lambda i: (i, 0)),
      out_shape=jax.ShapeDtypeStruct((n, DIM), jnp.float32),
  )(emb, w_enc, b_enc.reshape(1, DIM), w_dec, b_dec.reshape(1, DIM))


def kernel(x, table, W_enc, b_enc, W_dec, b_dec):
  b, s = x.shape
  n = b * s
  idx = x.reshape(-1).astype(jnp.int32)
  ngroups = n // (NUM_WORKERS * GROUP)
  idx3 = idx.reshape(NUM_WORKERS, ngroups, GROUP)
  emb = _sc_gather(table, idx3)
  out = _tc_mlp(emb, W_enc, b_enc, W_dec, b_dec, blk=4096)
  return out.reshape(b, s, DIM)


# speedup vs baseline: 1.0910x; 1.0910x over previous
import functools

import jax
import jax.numpy as jnp
from jax import lax
from jax.experimental import pallas as pl
from jax.experimental.pallas import tpu as pltpu
from jax.experimental.pallas import tpu_sc as plsc

DIM = 64
GROUP = 128
NUM_WORKERS = 32


K = 8


def _gather_body(table_hbm, idx_hbm, out_hbm, idx_v, rows_v, gsem, ssem):
  ngroups = idx_hbm.shape[1]
  nchunks = ngroups // K
  wid = lax.axis_index("s") * 2 + lax.axis_index("c")
  pltpu.sync_copy(idx_hbm.at[wid], idx_v)
  base = wid * (ngroups * GROUP)

  def gather(g, b):
    return pltpu.make_async_copy(
        table_hbm.at[idx_v.at[g]], rows_v.at[b], gsem.at[b])

  def scatter(g, b):
    return pltpu.make_async_copy(
        rows_v.at[b], out_hbm.at[pl.ds(base + g * GROUP, GROUP)], ssem.at[b])

  for b in range(K):
    gather(b, b).start()

  def outer(c, carry):
    g0 = c * K
    for b in range(K):
      gather(g0 + b, b).wait()
      scatter(g0 + b, b).start()
    for b in range(K):
      scatter(g0 + b, b).wait()

      @pl.when(c < nchunks - 1)
      def _():
        gather(g0 + K + b, b).start()

    return carry

  lax.fori_loop(0, nchunks, outer, 0)


def _sc_gather(table, idx3):
  nw, ngroups, group = idx3.shape
  n = nw * ngroups * group
  mesh = plsc.VectorSubcoreMesh(core_axis_name="c", subcore_axis_name="s")
  return pl.kernel(
      _gather_body,
      out_type=jax.ShapeDtypeStruct((n, DIM), jnp.float32),
      mesh=mesh,
      scratch_types=[
          pltpu.VMEM((ngroups, group), jnp.int32),
          pltpu.VMEM((K, group, DIM), jnp.float32),
          pltpu.SemaphoreType.DMA((K,)),
          pltpu.SemaphoreType.DMA((K,)),
      ],
      compiler_params=pltpu.CompilerParams(use_tc_tiling_on_sc=False),
  )(table, idx3)


def _mlp_body(emb_ref, we_ref, be_ref, wd_ref, bd_ref, out_ref):
  h = jnp.dot(emb_ref[...], we_ref[...], preferred_element_type=jnp.float32)
  h = jnp.maximum(h + be_ref[...], 0.0)
  out = jnp.dot(h, wd_ref[...], preferred_element_type=jnp.float32)
  out_ref[...] = out + bd_ref[...]


def _tc_mlp(emb, w_enc, b_enc, w_dec, b_dec, blk):
  n = emb.shape[0]
  return pl.pallas_call(
      _mlp_body,
      grid=(n // blk,),
      in_specs=[
          pl.BlockSpec((blk, DIM), lambda i: (i, 0)),
          pl.BlockSpec((DIM, DIM), lambda i: (0, 0)),
          pl.BlockSpec((1, DIM), lambda i: (0, 0)),
          pl.BlockSpec((DIM, DIM), lambda i: (0, 0)),
          pl.BlockSpec((1, DIM), lambda i: (0, 0)),
      ],
      out_specs=pl.BlockSpec((blk, DIM), lambda i: (i, 0)),
      out_shape=jax.ShapeDtypeStruct((n, DIM), jnp.float32),
  )(emb, w_enc, b_enc.reshape(1, DIM), w_dec, b_dec.reshape(1, DIM))


def kernel(x, table, W_enc, b_enc, W_dec, b_dec):
  b, s = x.shape
  n = b * s
  idx = x.reshape(-1).astype(jnp.int32)
  ngroups = n // (NUM_WORKERS * GROUP)
  idx3 = idx.reshape(NUM_WORKERS, ngroups, GROUP)
  emb = _sc_gather(table, idx3)
  out = _tc_mlp(emb, W_enc, b_enc, W_dec, b_dec, blk=4096)
  return out.reshape(b, s, DIM)


# speedup vs baseline: 2.6143x; 2.3961x over previous
import jax
import jax.numpy as jnp
from jax import lax
from jax.experimental import pallas as pl
from jax.experimental.pallas import tpu as pltpu
from jax.experimental.pallas import tpu_sc as plsc

DIM = 64
GROUP = 128
NUM_WORKERS = 32
K = 8
BLK = 8192


def _precompute_body(tbl_ref, we_ref, be_ref, wd_ref, bd_ref, out_ref):
  x = tbl_ref[...]
  h = lax.dot_general(x, we_ref[...], (((0,), (0,)), ((), ())),
                      preferred_element_type=jnp.float32)
  h = jnp.maximum(h + be_ref[...], 0.0)
  y = lax.dot_general(h, wd_ref[...], (((1,), (0,)), ((), ())),
                      preferred_element_type=jnp.float32)
  y = y + bd_ref[...]
  out_ref[...] = jnp.concatenate([y[:BLK // 2], y[BLK // 2:]], axis=-1)


def _tc_precompute(tbl_t, w_enc, b_enc, w_dec, b_dec):
  v = tbl_t.shape[1]
  nblocks = (v + BLK - 1) // BLK
  return pl.pallas_call(
      _precompute_body,
      grid=(nblocks,),
      in_specs=[
          pl.BlockSpec((DIM, BLK), lambda i: (0, i)),
          pl.BlockSpec((DIM, DIM), lambda i: (0, 0)),
          pl.BlockSpec((1, DIM), lambda i: (0, 0)),
          pl.BlockSpec((DIM, DIM), lambda i: (0, 0)),
          pl.BlockSpec((1, DIM), lambda i: (0, 0)),
      ],
      out_specs=pl.BlockSpec((BLK // 2, 2 * DIM), lambda i: (i, 0)),
      out_shape=jax.ShapeDtypeStruct((nblocks * (BLK // 2), 2 * DIM),
                                     jnp.float32),
  )(tbl_t, w_enc, b_enc.reshape(1, DIM), w_dec, b_dec.reshape(1, DIM))


def _gather_body(table_hbm, idx_hbm, didx_hbm, out_hbm, idx_v, didx_v, rows_v,
                 gsem, ssem):
  ngroups = idx_hbm.shape[1]
  nchunks = ngroups // K
  wid = lax.axis_index("s") * 2 + lax.axis_index("c")
  pltpu.sync_copy(idx_hbm.at[wid], idx_v)
  pltpu.sync_copy(didx_hbm.at[wid], didx_v)

  def gather(g, b):
    return pltpu.make_async_copy(
        table_hbm.at[idx_v.at[g]], rows_v.at[b], gsem.at[b])

  def scatter(g, b):
    return pltpu.make_async_copy(
        rows_v.at[b], out_hbm.at[didx_v.at[g]], ssem.at[b])

  for b in range(K):
    gather(b, b).start()

  def outer(c, carry):
    g0 = c * K
    for b in range(K):
      gather(g0 + b, b).wait()
      scatter(g0 + b, b).start()
    for b in range(K):
      scatter(g0 + b, b).wait()

      @pl.when(c < nchunks - 1)
      def _():
        gather(g0 + K + b, b).start()

    return carry

  lax.fori_loop(0, nchunks, outer, 0)


def _sc_gather(table, idx3, didx3):
  nw, ngroups, group = idx3.shape
  n = nw * ngroups * group
  mesh = plsc.VectorSubcoreMesh(core_axis_name="c", subcore_axis_name="s")
  return pl.kernel(
      _gather_body,
      out_type=jax.ShapeDtypeStruct((n, DIM), jnp.float32),
      mesh=mesh,
      scratch_types=[
          pltpu.VMEM((ngroups, group), jnp.int32),
          pltpu.VMEM((ngroups, group), jnp.int32),
          pltpu.VMEM((K, group, DIM), jnp.float32),
          pltpu.SemaphoreType.DMA((K,)),
          pltpu.SemaphoreType.DMA((K,)),
      ],
      compiler_params=pltpu.CompilerParams(use_tc_tiling_on_sc=False),
  )(table, idx3, didx3)


def _transpose_body(emb_ref, out_ref):
  x2 = emb_ref[0]
  xt = x2.T
  out_ref[0] = jnp.concatenate([xt[:DIM], xt[DIM:]], axis=-1)


def _tc_transpose(emb2, b, s):
  return pl.pallas_call(
      _transpose_body,
      grid=(s,),
      in_specs=[pl.BlockSpec((1, b // 2, 2 * DIM), lambda i: (i, 0, 0))],
      out_specs=pl.BlockSpec((1, DIM, b), lambda i: (i, 0, 0)),
      out_shape=jax.ShapeDtypeStruct((s, DIM, b), jnp.float32),
  )(emb2)


def kernel(x, table, W_enc, b_enc, W_dec, b_dec):
  b, s = x.shape
  n = b * s
  t4 = _tc_precompute(table.T, W_enc, b_enc, W_dec, b_dec)
  t4 = t4.reshape(t4.shape[0] * 2, DIM)
  idx = x.T.reshape(-1).astype(jnp.int32)
  i = idx // BLK
  c = idx % BLK
  idx = 2 * (i * (BLK // 2) + c % (BLK // 2)) + c // (BLK // 2)
  ngroups = n // (NUM_WORKERS * GROUP)
  idx3 = idx.reshape(NUM_WORKERS, ngroups, GROUP)
  j = jnp.arange(n, dtype=jnp.int32)
  bb = j % b
  didx = (j - bb) + 2 * (bb % (b // 2)) + bb // (b // 2)
  didx3 = didx.reshape(NUM_WORKERS, ngroups, GROUP)
  emb = _sc_gather(t4, idx3, didx3)
  out_t = _tc_transpose(emb.reshape(s, b // 2, 2 * DIM), b, s)
  return jnp.transpose(out_t, (2, 0, 1))
